# baseline (speedup 1.0000x reference)
import jax
import jax.numpy as jnp
from jax import lax
from jax.experimental import pallas as pl
from jax.experimental.pallas import tpu as pltpu
from jax.experimental.pallas import tpu_sc as plsc

B = 16384
D = 128
NC = 2
NS = 16
NW = NC * NS
BPW = B // NW
CH = 128
NCH = BPW // CH
L = 16


def _sc_body(tri_hbm, ent_hbm, rel_hbm, out_hbm,
             idx_v,
             hrow0, rrow0, trow0, hrow1, rrow1, trow1,
             score_v, sem0, sem1):
    wid = lax.axis_index("s") * NC + lax.axis_index("c")
    base = wid * BPW
    pltpu.sync_copy(tri_hbm.at[pl.ds(3 * base, 3 * BPW)], idx_v)
    lane = lax.iota(jnp.int32, L)

    bufs = ((hrow0, rrow0, trow0, sem0), (hrow1, rrow1, trow1, sem1))

    def issue(k):
        hb, rb, tb, sem = bufs[k % 2]
        return (
            pltpu.async_copy(ent_hbm.at[idx_v.at[pl.ds(k * CH, CH)]],
                             hb, sem),
            pltpu.async_copy(rel_hbm.at[idx_v.at[pl.ds(BPW + k * CH, CH)]],
                             rb, sem),
            pltpu.async_copy(ent_hbm.at[idx_v.at[pl.ds(2 * BPW + k * CH, CH)]],
                             tb, sem),
        )

    def compute(k):
        hb, rb, tb, _ = bufs[k % 2]

        def group_body(g, carry2):
            def triple_body(i, res):
                row = g * L + i
                acc0 = jnp.zeros((L,), jnp.float32)
                acc1 = jnp.zeros((L,), jnp.float32)
                for c in range(0, D // L, 2):
                    h0 = hb[row, pl.ds(c * L, L)]
                    r0 = rb[row, pl.ds(c * L, L)]
                    t0 = tb[row, pl.ds(c * L, L)]
                    acc0 = acc0 + jnp.abs(h0 + r0 - t0)
                    h1 = hb[row, pl.ds((c + 1) * L, L)]
                    r1 = rb[row, pl.ds((c + 1) * L, L)]
                    t1 = tb[row, pl.ds((c + 1) * L, L)]
                    acc1 = acc1 + jnp.abs(h1 + r1 - t1)
                s = jnp.sum(acc0 + acc1)
                return jnp.where(lane == i, s, res)

            res = lax.fori_loop(0, L, triple_body,
                                jnp.zeros((L,), jnp.float32))
            score_v[pl.ds(k * CH + g * L, L)] = -res
            return carry2

        lax.fori_loop(0, CH // L, group_body, 0)

    pending = issue(0)
    for k in range(NCH):
        for cp in pending:
            cp.wait()
        if k + 1 < NCH:
            pending = issue(k + 1)
        compute(k)
    pltpu.sync_copy(score_v, out_hbm.at[pl.ds(base, BPW)])


@jax.jit
def kernel(triples, ent_emb, rel_emb):
    tri = triples.reshape(NW, BPW, 3).transpose(0, 2, 1).reshape(3 * B)
    mesh = plsc.VectorSubcoreMesh(core_axis_name="c", subcore_axis_name="s")
    run = pl.kernel(
        _sc_body,
        out_type=jax.ShapeDtypeStruct((B,), jnp.float32),
        mesh=mesh,
        compiler_params=pltpu.CompilerParams(needs_layout_passes=False),
        scratch_types=[
            pltpu.VMEM((3 * BPW,), jnp.int32),
            pltpu.VMEM((CH, D), jnp.float32),
            pltpu.VMEM((CH, D), jnp.float32),
            pltpu.VMEM((CH, D), jnp.float32),
            pltpu.VMEM((CH, D), jnp.float32),
            pltpu.VMEM((CH, D), jnp.float32),
            pltpu.VMEM((CH, D), jnp.float32),
            pltpu.VMEM((BPW,), jnp.float32),
            pltpu.SemaphoreType.DMA,
            pltpu.SemaphoreType.DMA,
        ],
    )
    return run(tri, ent_emb, rel_emb)
